# baseline (speedup 1.0000x reference)
import functools

import jax
import jax.numpy as jnp
from jax import lax
from jax.experimental import pallas as pl
from jax.experimental.pallas import tpu as pltpu
from jax.experimental.pallas import tpu_sc as plsc

_VOCAB = 1000000
_EMBD = 32
_BATCH = 16384
_NNEG = 5
_PACK = 128 // _EMBD

_NC = 2
_NS = 16
_L = 16
_NW = _NC * _NS
_BPW = _BATCH // _NW
_CH = 128
_NCHUNK = _BPW // _CH
_GPC = _CH // _L


def _sc_scores_body(u_row_hbm, p_row_hbm, n_row_hbm,
                    u_rem_hbm, p_rem_hbm, n_rem_hbm,
                    U_hbm, V_hbm,
                    pos_out, neg_out,
                    urowi, prowi, nrowi, urem, prem, nrem,
                    urows, prows, nrows, psc, nsc, sem):
    wid = lax.axis_index("s") * _NC + lax.axis_index("c")
    base = wid * _BPW

    pltpu.sync_copy(u_row_hbm.at[pl.ds(base, _BPW)], urowi)
    pltpu.sync_copy(p_row_hbm.at[pl.ds(base, _BPW)], prowi)
    pltpu.sync_copy(n_row_hbm.at[pl.ds(base * _NNEG, _BPW * _NNEG)], nrowi)
    pltpu.sync_copy(u_rem_hbm.at[pl.ds(base, _BPW)], urem)
    pltpu.sync_copy(p_rem_hbm.at[pl.ds(base, _BPW)], prem)
    pltpu.sync_copy(n_rem_hbm.at[pl.ds(base * _NNEG, _BPW * _NNEG)], nrem)

    iot = lax.iota(jnp.int32, _L)

    for c in range(_NCHUNK):
        copies = [
            pltpu.async_copy(
                U_hbm.at[urowi.at[pl.ds(c * _CH, _CH)]], urows, sem),
            pltpu.async_copy(
                V_hbm.at[prowi.at[pl.ds(c * _CH, _CH)]], prows, sem),
        ]
        for j in range(_NNEG):
            copies.append(pltpu.async_copy(
                V_hbm.at[nrowi.at[pl.ds((c * _NNEG + j) * _CH, _CH)]],
                nrows.at[pl.ds(j * _CH, _CH)], sem))
        for cp in copies:
            cp.wait()

        def group(g, carry):
            rb = g * _L + iot
            grow = c * _CH + rb
            ucol0 = urem[pl.ds(c * _CH + g * _L, _L)] * _EMBD
            pcol0 = prem[pl.ds(c * _CH + g * _L, _L)] * _EMBD
            ncol0 = [plsc.load_gather(nrem, [grow * _NNEG + n]) * _EMBD
                     for n in range(_NNEG)]
            pos_acc = jnp.zeros((_L,), jnp.float32)
            neg_acc = jnp.zeros((_L,), jnp.float32)
            for d in range(_EMBD):
                uc = plsc.load_gather(urows, [rb, ucol0 + d])
                pc = plsc.load_gather(prows, [rb, pcol0 + d])
                nsum = plsc.load_gather(nrows, [rb * _NNEG, ncol0[0] + d])
                for n in range(1, _NNEG):
                    nsum = nsum + plsc.load_gather(
                        nrows, [rb * _NNEG + n, ncol0[n] + d])
                pos_acc = pos_acc + uc * pc
                neg_acc = neg_acc + uc * nsum
            psc[pl.ds(c * _CH + g * _L, _L)] = pos_acc
            nsc[pl.ds(c * _CH + g * _L, _L)] = neg_acc
            return carry

        lax.fori_loop(0, _GPC, group, 0)

    pltpu.sync_copy(psc, pos_out.at[pl.ds(base, _BPW)])
    pltpu.sync_copy(nsc, neg_out.at[pl.ds(base, _BPW)])


_sc_scores = functools.partial(
    pl.kernel,
    out_type=[jax.ShapeDtypeStruct((_BATCH,), jnp.float32),
              jax.ShapeDtypeStruct((_BATCH,), jnp.float32)],
    mesh=plsc.VectorSubcoreMesh(core_axis_name="c", subcore_axis_name="s"),
    compiler_params=pltpu.CompilerParams(needs_layout_passes=False),
    scratch_types=[
        pltpu.VMEM((_BPW,), jnp.int32),
        pltpu.VMEM((_BPW,), jnp.int32),
        pltpu.VMEM((_BPW * _NNEG,), jnp.int32),
        pltpu.VMEM((_BPW,), jnp.int32),
        pltpu.VMEM((_BPW,), jnp.int32),
        pltpu.VMEM((_BPW * _NNEG,), jnp.int32),
        pltpu.VMEM((_CH, 128), jnp.float32),
        pltpu.VMEM((_CH, 128), jnp.float32),
        pltpu.VMEM((_CH * _NNEG, 128), jnp.float32),
        pltpu.VMEM((_BPW,), jnp.float32),
        pltpu.VMEM((_BPW,), jnp.float32),
        pltpu.SemaphoreType.DMA,
    ],
)(_sc_scores_body)


def _tc_final_body(pos_ref, neg_ref, out_ref):
    pos = pos_ref[...]
    neg = neg_ref[...]
    total = (jnp.sum(jax.nn.log_sigmoid(pos))
             + jnp.sum(jax.nn.log_sigmoid(-neg)))
    out_ref[0, 0] = -total / _BATCH


def _tc_final(pos2d, neg2d):
    return pl.pallas_call(
        _tc_final_body,
        out_shape=jax.ShapeDtypeStruct((1, 1), jnp.float32),
        out_specs=pl.BlockSpec(memory_space=pltpu.SMEM),
    )(pos2d, neg2d)


def kernel(u_pos, v_pos, v_neg, U, V):
    u = u_pos.astype(jnp.int32)
    p = v_pos.astype(jnp.int32)
    n = v_neg.astype(jnp.int32).reshape(_BATCH * _NNEG)
    U128 = U.reshape(_VOCAB // _PACK, 128)
    V128 = V.reshape(_VOCAB // _PACK, 128)
    pos, neg = _sc_scores(
        u >> 2, p >> 2, n >> 2, u & 3, p & 3, n & 3, U128, V128)
    res = _tc_final(pos.reshape(_BATCH // 128, 128),
                    neg.reshape(_BATCH // 128, 128))
    return res[0, 0]
